# Initial kernel scaffold; baseline (speedup 1.0000x reference)
import jax
import jax.numpy as jnp
from jax.experimental import pallas as pl


def kernel(x_features, x_edge_idx, x_edge_attr, W_gcn, b_gcn, W_fc, b_fc):
    raise NotImplementedError("write your pallas kernel here")


# speedup vs baseline: 4.1596x; 4.1596x over previous
import jax
import jax.numpy as jnp
from jax import lax
from jax.experimental import pallas as pl
from jax.experimental.pallas import tpu as pltpu
from jax.experimental.pallas import tpu_sc as plsc

N = 10000
E = 160000
D = 256
H = 210
O = 128

NP_ = 10240
HP = 256
HH = 128
NTILES = 16
NODES_PER_TILE = NP_ // NTILES
ECHUNK = 128
E2 = E + N
NCHUNK = 88
SLAB = 8
NSLAB = NCHUNK // SLAB
EPT = NCHUNK * ECHUNK
E2P = EPT * NTILES

_SC_MESH = plsc.VectorSubcoreMesh(core_axis_name="c", subcore_axis_name="s")


def _deg_body(col_hbm, ew_hbm, deg_hbm, colv, ewv, degv, deg_sh, sem):
  c = lax.axis_index("c")
  s = lax.axis_index("s")

  @pl.when(c == 0)
  def _():
    def _z(i):
      degv[pl.ds(i * 16, 16)] = jnp.zeros((16,), jnp.float32)
    pl.loop(0, NODES_PER_TILE // 16)(_z)
    pltpu.sync_copy(degv, deg_sh.at[pl.ds(s * NODES_PER_TILE, NODES_PER_TILE)])
    plsc.subcore_barrier()

    pltpu.sync_copy(col_hbm.at[s], colv)
    pltpu.sync_copy(ew_hbm.at[s], ewv)

    def _sc(g):
      pltpu.async_copy(ewv.at[g], deg_sh.at[colv.at[g]], sem, add=True).wait()
    pl.loop(0, NCHUNK)(_sc)
    plsc.subcore_barrier()

    pltpu.sync_copy(deg_sh.at[pl.ds(s * NODES_PER_TILE, NODES_PER_TILE)],
                    deg_hbm.at[pl.ds(s * NODES_PER_TILE, NODES_PER_TILE)])


def _phase_a(col2, ew2):
  return pl.kernel(
      _deg_body,
      out_type=jax.ShapeDtypeStruct((NP_,), jnp.float32),
      mesh=_SC_MESH,
      scratch_types=[
          pltpu.VMEM((NCHUNK, ECHUNK), jnp.int32),
          pltpu.VMEM((NCHUNK, ECHUNK), jnp.float32),
          pltpu.VMEM((NODES_PER_TILE,), jnp.float32),
          pltpu.VMEM_SHARED((NP_,), jnp.float32),
          pltpu.SemaphoreType.DMA,
      ],
  )(col2, ew2)


def _xws_body(x_ref, w_ref, deg_ref, o0_ref, o1_ref):
  xw = jnp.dot(x_ref[...], w_ref[...], preferred_element_type=jnp.float32)
  deg = deg_ref[...]
  dis = jnp.where(deg > 0.0, lax.rsqrt(jnp.maximum(deg, 1e-12)), 0.0)
  xws = xw * dis
  o0_ref[...] = xws[:, :HH]
  o1_ref[...] = xws[:, HH:]


def _phase_b(xpad, wpad, deg):
  blk = 256
  return pl.pallas_call(
      _xws_body,
      grid=(NP_ // blk,),
      in_specs=[
          pl.BlockSpec((blk, D), lambda i: (i, 0)),
          pl.BlockSpec((D, HP), lambda i: (0, 0)),
          pl.BlockSpec((blk, 1), lambda i: (i, 0)),
      ],
      out_specs=[
          pl.BlockSpec((blk, HH), lambda i: (i, 0)),
          pl.BlockSpec((blk, HH), lambda i: (i, 0)),
      ],
      out_shape=[
          jax.ShapeDtypeStruct((NP_, HH), jnp.float32),
          jax.ShapeDtypeStruct((NP_, HH), jnp.float32),
      ],
  )(xpad, wpad, deg.reshape(NP_, 1))


def _edge_body(row_hbm, col_hbm, ew_hbm, xws0_hbm, xws1_hbm, out_hbm,
               rowv, colv, ewv, rows, acc_sh, sem):
  c = lax.axis_index("c")
  s = lax.axis_index("s")

  def _z(e):
    for j in range(HH // 16):
      rows[e, pl.ds(j * 16, 16)] = jnp.zeros((16,), jnp.float32)
  pl.loop(0, ECHUNK)(_z)

  def _zc(i):
    pltpu.sync_copy(
        rows, acc_sh.at[pl.ds(s * NODES_PER_TILE + i * ECHUNK, ECHUNK)])
  pl.loop(0, NODES_PER_TILE // ECHUNK)(_zc)

  plsc.subcore_barrier()

  def _chunks(src_hbm):
    def _slab(t):
      pltpu.sync_copy(row_hbm.at[s, pl.ds(t * SLAB, SLAB)], rowv)
      pltpu.sync_copy(col_hbm.at[s, pl.ds(t * SLAB, SLAB)], colv)
      pltpu.sync_copy(ew_hbm.at[s, pl.ds(t * SLAB, SLAB)], ewv)

      def _g(g):
        pltpu.async_copy(src_hbm.at[rowv.at[g]], rows, sem).wait()
        def _eb(eb):
          wv = ewv[g, pl.ds(eb * 16, 16)]
          for k in range(16):
            e = eb * 16 + k
            w = wv[k]
            for j in range(HH // 16):
              rows[e, pl.ds(j * 16, 16)] = rows[e, pl.ds(j * 16, 16)] * w
        pl.loop(0, ECHUNK // 16)(_eb)
        pltpu.async_copy(rows, acc_sh.at[colv.at[g]], sem, add=True).wait()
      pl.loop(0, SLAB)(_g)
    pl.loop(0, NSLAB)(_slab)

  @pl.when(c == 0)
  def _():
    _chunks(xws0_hbm)

  @pl.when(c == 1)
  def _():
    _chunks(xws1_hbm)

  plsc.subcore_barrier()

  nbase = s * NODES_PER_TILE

  @pl.when(c == 0)
  def _():
    pltpu.sync_copy(acc_sh.at[pl.ds(nbase, NODES_PER_TILE)],
                    out_hbm.at[0, pl.ds(nbase, NODES_PER_TILE)])

  @pl.when(c == 1)
  def _():
    pltpu.sync_copy(acc_sh.at[pl.ds(nbase, NODES_PER_TILE)],
                    out_hbm.at[1, pl.ds(nbase, NODES_PER_TILE)])


def _phase_c(row2, col2, ew2, xws0, xws1):
  return pl.kernel(
      _edge_body,
      out_type=jax.ShapeDtypeStruct((2, NP_, HH), jnp.float32),
      mesh=_SC_MESH,
      scratch_types=[
          pltpu.VMEM((SLAB, ECHUNK), jnp.int32),
          pltpu.VMEM((SLAB, ECHUNK), jnp.int32),
          pltpu.VMEM((SLAB, ECHUNK), jnp.float32),
          pltpu.VMEM((ECHUNK, HH), jnp.float32),
          pltpu.VMEM_SHARED((NP_, HH), jnp.float32),
          pltpu.SemaphoreType.DMA,
      ],
  )(row2, col2, ew2, xws0, xws1)


def _fc_body(a0_ref, a1_ref, deg_ref, w0_ref, w1_ref, b_ref, o_ref):
  deg = deg_ref[...]
  d = jnp.where(deg > 0.0, lax.rsqrt(jnp.maximum(deg, 1e-12)), 0.0)
  h = jnp.dot(a0_ref[...] * d, w0_ref[...], preferred_element_type=jnp.float32)
  h = h + jnp.dot(a1_ref[...] * d, w1_ref[...],
                  preferred_element_type=jnp.float32)
  o_ref[...] = jnp.tanh(h + b_ref[...])


def _phase_d(acc0, acc1, deg, wf0, wf1, beff):
  blk = 256
  return pl.pallas_call(
      _fc_body,
      grid=(NP_ // blk,),
      in_specs=[
          pl.BlockSpec((blk, HH), lambda i: (i, 0)),
          pl.BlockSpec((blk, HH), lambda i: (i, 0)),
          pl.BlockSpec((blk, 1), lambda i: (i, 0)),
          pl.BlockSpec((HH, O), lambda i: (0, 0)),
          pl.BlockSpec((HH, O), lambda i: (0, 0)),
          pl.BlockSpec((1, O), lambda i: (0, 0)),
      ],
      out_specs=pl.BlockSpec((blk, O), lambda i: (i, 0)),
      out_shape=jax.ShapeDtypeStruct((NP_, O), jnp.float32),
  )(acc0, acc1, deg.reshape(NP_, 1), wf0, wf1, beff.reshape(1, O))


@jax.jit
def kernel(x_features, x_edge_idx, x_edge_attr, W_gcn, b_gcn, W_fc, b_fc):
  row = x_edge_idx[0].astype(jnp.int32)
  col = x_edge_idx[1].astype(jnp.int32)
  loop = jnp.arange(N, dtype=jnp.int32)
  pad = E2P - E2
  row2 = jnp.concatenate([row, loop, jnp.zeros((pad,), jnp.int32)])
  col2 = jnp.concatenate([col, loop, jnp.zeros((pad,), jnp.int32)])
  ew2 = jnp.concatenate([x_edge_attr.astype(jnp.float32),
                         jnp.ones((N,), jnp.float32),
                         jnp.zeros((pad,), jnp.float32)])
  row2 = row2.reshape(NTILES, NCHUNK, ECHUNK)
  col2 = col2.reshape(NTILES, NCHUNK, ECHUNK)
  ew2 = ew2.reshape(NTILES, NCHUNK, ECHUNK)

  deg = _phase_a(col2, ew2)

  xpad = jnp.zeros((NP_, D), jnp.float32).at[:N].set(x_features)
  wpad = jnp.zeros((D, HP), jnp.float32).at[:, :H].set(W_gcn)
  xws0, xws1 = _phase_b(xpad, wpad, deg)

  acc = _phase_c(row2, col2, ew2, xws0, xws1)

  wfpad = jnp.zeros((HP, O), jnp.float32).at[:H].set(W_fc)
  beff = b_gcn @ W_fc + b_fc
  out = _phase_d(acc[0], acc[1], deg, wfpad[:HH], wfpad[HH:], beff)
  return out[:N]


# speedup vs baseline: 4.8019x; 1.1544x over previous
import jax
import jax.numpy as jnp
from jax import lax
from jax.experimental import pallas as pl
from jax.experimental.pallas import tpu as pltpu
from jax.experimental.pallas import tpu_sc as plsc

N = 10000
E = 160000
D = 256
H = 210
O = 128

NP_ = 10240
HP = 256
HH = 128
NTILES = 16
NODES_PER_TILE = NP_ // NTILES
ECHUNK = 96
E2 = E + N
NCHUNK = 120
SLAB = 8
NSLAB = NCHUNK // SLAB
EPT = NCHUNK * ECHUNK
E2P = EPT * NTILES

_SC_MESH = plsc.VectorSubcoreMesh(core_axis_name="c", subcore_axis_name="s")


def _deg_body(col_hbm, ew_hbm, deg_hbm, colv, ewv, degv, deg_sh, sem):
  c = lax.axis_index("c")
  s = lax.axis_index("s")

  @pl.when(c == 0)
  def _():
    def _z(i):
      degv[pl.ds(i * 16, 16)] = jnp.zeros((16,), jnp.float32)
    pl.loop(0, NODES_PER_TILE // 16)(_z)
    pltpu.sync_copy(degv, deg_sh.at[pl.ds(s * NODES_PER_TILE, NODES_PER_TILE)])
    plsc.subcore_barrier()

    pltpu.sync_copy(col_hbm.at[s], colv)
    pltpu.sync_copy(ew_hbm.at[s], ewv)

    def _sc(g):
      pltpu.async_copy(ewv.at[g], deg_sh.at[colv.at[g]], sem, add=True).wait()
    pl.loop(0, NCHUNK)(_sc)
    plsc.subcore_barrier()

    pltpu.sync_copy(deg_sh.at[pl.ds(s * NODES_PER_TILE, NODES_PER_TILE)],
                    deg_hbm.at[pl.ds(s * NODES_PER_TILE, NODES_PER_TILE)])


def _phase_a(col2, ew2):
  return pl.kernel(
      _deg_body,
      out_type=jax.ShapeDtypeStruct((NP_,), jnp.float32),
      mesh=_SC_MESH,
      scratch_types=[
          pltpu.VMEM((NCHUNK, ECHUNK), jnp.int32),
          pltpu.VMEM((NCHUNK, ECHUNK), jnp.float32),
          pltpu.VMEM((NODES_PER_TILE,), jnp.float32),
          pltpu.VMEM_SHARED((NP_,), jnp.float32),
          pltpu.SemaphoreType.DMA,
      ],
  )(col2, ew2)


def _xws_body(x_ref, w_ref, deg_ref, o0_ref, o1_ref):
  xw = jnp.dot(x_ref[...], w_ref[...], preferred_element_type=jnp.float32)
  deg = deg_ref[...]
  dis = jnp.where(deg > 0.0, lax.rsqrt(jnp.maximum(deg, 1e-12)), 0.0)
  xws = xw * dis
  o0_ref[...] = xws[:, :HH]
  o1_ref[...] = xws[:, HH:]


def _phase_b(xpad, wpad, deg):
  blk = 256
  return pl.pallas_call(
      _xws_body,
      grid=(NP_ // blk,),
      in_specs=[
          pl.BlockSpec((blk, D), lambda i: (i, 0)),
          pl.BlockSpec((D, HP), lambda i: (0, 0)),
          pl.BlockSpec((blk, 1), lambda i: (i, 0)),
      ],
      out_specs=[
          pl.BlockSpec((blk, HH), lambda i: (i, 0)),
          pl.BlockSpec((blk, HH), lambda i: (i, 0)),
      ],
      out_shape=[
          jax.ShapeDtypeStruct((NP_, HH), jnp.float32),
          jax.ShapeDtypeStruct((NP_, HH), jnp.float32),
      ],
  )(xpad, wpad, deg.reshape(NP_, 1))


def _edge_body(row_hbm, col_hbm, ew_hbm, xws0_hbm, xws1_hbm, out_hbm,
               rowv, colv, ewv, b0, b1, b2, acc_sh,
               gs0, gs1, gs2, ss0, ss1, ss2):
  c = lax.axis_index("c")
  s = lax.axis_index("s")
  bufs = (b0, b1, b2)
  gsems = (gs0, gs1, gs2)
  ssems = (ss0, ss1, ss2)

  def _z(e):
    for j in range(HH // 16):
      b0[e, pl.ds(j * 16, 16)] = jnp.zeros((16,), jnp.float32)
  pl.loop(0, ECHUNK)(_z)

  def _zc(i):
    pltpu.sync_copy(
        b0, acc_sh.at[pl.ds(s * NODES_PER_TILE + i * ECHUNK, ECHUNK)])
  pl.loop(0, NODES_PER_TILE // ECHUNK)(_zc)
  pltpu.sync_copy(
      b0.at[pl.ds(0, NODES_PER_TILE % ECHUNK)],
      acc_sh.at[pl.ds(
          s * NODES_PER_TILE + (NODES_PER_TILE // ECHUNK) * ECHUNK,
          NODES_PER_TILE % ECHUNK)])

  plsc.subcore_barrier()

  def _chunks(src_hbm):
    pltpu.sync_copy(row_hbm.at[s, pl.ds(0, SLAB)], rowv.at[0])
    pltpu.async_copy(src_hbm.at[rowv.at[0, 0]], b0, gs0)
    pltpu.async_copy(src_hbm.at[rowv.at[0, 1]], b1, gs1)

    def _body(g):
      for k in range(3):
        c = g + k
        t = c // SLAB
        k8 = c - t * SLAB
        tp = lax.rem(t, 2)

        @pl.when(k8 == 0)
        def _stage():
          pltpu.sync_copy(col_hbm.at[s, pl.ds(t * SLAB, SLAB)], colv.at[tp])
          pltpu.sync_copy(ew_hbm.at[s, pl.ds(t * SLAB, SLAB)], ewv)

          @pl.when(t + 1 < NSLAB)
          def _():
            pltpu.sync_copy(row_hbm.at[s, pl.ds((t + 1) * SLAB, SLAB)],
                            rowv.at[lax.rem(t + 1, 2)])

        pltpu.make_async_copy(src_hbm.at[rowv.at[0, 0]], bufs[k],
                              gsems[k]).wait()

        def _eb(eb):
          wv = ewv[k8, pl.ds(eb * 16, 16)]
          for kk in range(16):
            e = eb * 16 + kk
            w = wv[kk]
            for j in range(HH // 16):
              bufs[k][e, pl.ds(j * 16, 16)] = (
                  bufs[k][e, pl.ds(j * 16, 16)] * w)
        pl.loop(0, ECHUNK // 16)(_eb)

        nb = (k + 2) % 3

        @pl.when(c + 2 < NCHUNK)
        def _():
          @pl.when(c >= 1)
          def _():
            pltpu.make_async_copy(bufs[nb], acc_sh.at[colv.at[0, 0]],
                                  ssems[nb]).wait()
          c2 = c + 2
          t2 = c2 // SLAB
          pltpu.async_copy(
              src_hbm.at[rowv.at[lax.rem(t2, 2), c2 - t2 * SLAB]],
              bufs[nb], gsems[nb])

        pltpu.async_copy(bufs[k], acc_sh.at[colv.at[tp, k8]], ssems[k],
                         add=True)

    pl.loop(0, NCHUNK, step=3)(_body)

    for k in range(3):
      pltpu.make_async_copy(bufs[k], acc_sh.at[colv.at[0, 0]],
                            ssems[k]).wait()

  @pl.when(c == 0)
  def _():
    _chunks(xws0_hbm)

  @pl.when(c == 1)
  def _():
    _chunks(xws1_hbm)

  plsc.subcore_barrier()

  nbase = s * NODES_PER_TILE

  @pl.when(c == 0)
  def _():
    pltpu.sync_copy(acc_sh.at[pl.ds(nbase, NODES_PER_TILE)],
                    out_hbm.at[0, pl.ds(nbase, NODES_PER_TILE)])

  @pl.when(c == 1)
  def _():
    pltpu.sync_copy(acc_sh.at[pl.ds(nbase, NODES_PER_TILE)],
                    out_hbm.at[1, pl.ds(nbase, NODES_PER_TILE)])


def _phase_c(row2, col2, ew2, xws0, xws1):
  return pl.kernel(
      _edge_body,
      out_type=jax.ShapeDtypeStruct((2, NP_, HH), jnp.float32),
      mesh=_SC_MESH,
      scratch_types=[
          pltpu.VMEM((2, SLAB, ECHUNK), jnp.int32),
          pltpu.VMEM((2, SLAB, ECHUNK), jnp.int32),
          pltpu.VMEM((SLAB, ECHUNK), jnp.float32),
          pltpu.VMEM((ECHUNK, HH), jnp.float32),
          pltpu.VMEM((ECHUNK, HH), jnp.float32),
          pltpu.VMEM((ECHUNK, HH), jnp.float32),
          pltpu.VMEM_SHARED((NP_, HH), jnp.float32),
          pltpu.SemaphoreType.DMA,
          pltpu.SemaphoreType.DMA,
          pltpu.SemaphoreType.DMA,
          pltpu.SemaphoreType.DMA,
          pltpu.SemaphoreType.DMA,
          pltpu.SemaphoreType.DMA,
      ],
  )(row2, col2, ew2, xws0, xws1)


def _fc_body(a0_ref, a1_ref, deg_ref, w0_ref, w1_ref, b_ref, o_ref):
  deg = deg_ref[...]
  d = jnp.where(deg > 0.0, lax.rsqrt(jnp.maximum(deg, 1e-12)), 0.0)
  h = jnp.dot(a0_ref[...] * d, w0_ref[...], preferred_element_type=jnp.float32)
  h = h + jnp.dot(a1_ref[...] * d, w1_ref[...],
                  preferred_element_type=jnp.float32)
  o_ref[...] = jnp.tanh(h + b_ref[...])


def _phase_d(acc0, acc1, deg, wf0, wf1, beff):
  blk = 256
  return pl.pallas_call(
      _fc_body,
      grid=(NP_ // blk,),
      in_specs=[
          pl.BlockSpec((blk, HH), lambda i: (i, 0)),
          pl.BlockSpec((blk, HH), lambda i: (i, 0)),
          pl.BlockSpec((blk, 1), lambda i: (i, 0)),
          pl.BlockSpec((HH, O), lambda i: (0, 0)),
          pl.BlockSpec((HH, O), lambda i: (0, 0)),
          pl.BlockSpec((1, O), lambda i: (0, 0)),
      ],
      out_specs=pl.BlockSpec((blk, O), lambda i: (i, 0)),
      out_shape=jax.ShapeDtypeStruct((NP_, O), jnp.float32),
  )(acc0, acc1, deg.reshape(NP_, 1), wf0, wf1, beff.reshape(1, O))


@jax.jit
def kernel(x_features, x_edge_idx, x_edge_attr, W_gcn, b_gcn, W_fc, b_fc):
  row = x_edge_idx[0].astype(jnp.int32)
  col = x_edge_idx[1].astype(jnp.int32)
  loop = jnp.arange(N, dtype=jnp.int32)
  pad = E2P - E2
  row2 = jnp.concatenate([row, loop, jnp.zeros((pad,), jnp.int32)])
  col2 = jnp.concatenate([col, loop, jnp.zeros((pad,), jnp.int32)])
  ew2 = jnp.concatenate([x_edge_attr.astype(jnp.float32),
                         jnp.ones((N,), jnp.float32),
                         jnp.zeros((pad,), jnp.float32)])
  row2 = row2.reshape(NTILES, NCHUNK, ECHUNK)
  col2 = col2.reshape(NTILES, NCHUNK, ECHUNK)
  ew2 = ew2.reshape(NTILES, NCHUNK, ECHUNK)

  deg = _phase_a(col2, ew2)

  xpad = jnp.zeros((NP_, D), jnp.float32).at[:N].set(x_features)
  wpad = jnp.zeros((D, HP), jnp.float32).at[:, :H].set(W_gcn)
  xws0, xws1 = _phase_b(xpad, wpad, deg)

  acc = _phase_c(row2, col2, ew2, xws0, xws1)

  wfpad = jnp.zeros((HP, O), jnp.float32).at[:H].set(W_fc)
  beff = b_gcn @ W_fc + b_fc
  out = _phase_d(acc[0], acc[1], deg, wfpad[:HH], wfpad[HH:], beff)
  return out[:N]


# speedup vs baseline: 9.6111x; 2.0015x over previous
import jax
import jax.numpy as jnp
from jax import lax
from jax.experimental import pallas as pl
from jax.experimental.pallas import tpu as pltpu
from jax.experimental.pallas import tpu_sc as plsc

N = 10000
E = 160000
D = 256
H = 210
O = 128

NP_ = 10240
HP = 256
HH = 128
NTILES = 16
NODES_PER_TILE = NP_ // NTILES
ECHUNK = 80
E2 = E + N
NCHUNK = 136
NBUF = 4
SLAB = 8
NSLAB = NCHUNK // SLAB
EPT = NCHUNK * ECHUNK
E2P = EPT * NTILES

_SC_MESH = plsc.VectorSubcoreMesh(core_axis_name="c", subcore_axis_name="s")


def _deg_body(col_hbm, ew_hbm, deg_hbm, colv, ewv, degv, deg_sh, sem):
  c = lax.axis_index("c")
  s = lax.axis_index("s")

  @pl.when(c == 0)
  def _():
    def _z(i):
      degv[pl.ds(i * 16, 16)] = jnp.zeros((16,), jnp.float32)
    pl.loop(0, NODES_PER_TILE // 16)(_z)
    pltpu.sync_copy(degv, deg_sh.at[pl.ds(s * NODES_PER_TILE, NODES_PER_TILE)])
    plsc.subcore_barrier()

    pltpu.sync_copy(col_hbm.at[s], colv)
    pltpu.sync_copy(ew_hbm.at[s], ewv)

    def _sc(g):
      pltpu.async_copy(ewv.at[g], deg_sh.at[colv.at[g]], sem, add=True).wait()
    pl.loop(0, NCHUNK)(_sc)
    plsc.subcore_barrier()

    pltpu.sync_copy(deg_sh.at[pl.ds(s * NODES_PER_TILE, NODES_PER_TILE)],
                    deg_hbm.at[pl.ds(s * NODES_PER_TILE, NODES_PER_TILE)])


def _phase_a(col2, ew2):
  return pl.kernel(
      _deg_body,
      out_type=jax.ShapeDtypeStruct((NP_,), jnp.float32),
      mesh=_SC_MESH,
      scratch_types=[
          pltpu.VMEM((NCHUNK, ECHUNK), jnp.int32),
          pltpu.VMEM((NCHUNK, ECHUNK), jnp.float32),
          pltpu.VMEM((NODES_PER_TILE,), jnp.float32),
          pltpu.VMEM_SHARED((NP_,), jnp.float32),
          pltpu.SemaphoreType.DMA,
      ],
  )(col2, ew2)


def _xws_body(x_ref, w_ref, deg_ref, o0_ref, o1_ref):
  xw = jnp.dot(x_ref[...], w_ref[...], preferred_element_type=jnp.float32)
  deg = deg_ref[...]
  dis = jnp.where(deg > 0.0, lax.rsqrt(jnp.maximum(deg, 1e-12)), 0.0)
  xws = xw * dis
  o0_ref[...] = xws[:, :HH]
  o1_ref[...] = xws[:, HH:]


def _phase_b(xpad, wpad, deg):
  blk = 256
  return pl.pallas_call(
      _xws_body,
      grid=(NP_ // blk,),
      in_specs=[
          pl.BlockSpec((blk, D), lambda i: (i, 0)),
          pl.BlockSpec((D, HP), lambda i: (0, 0)),
          pl.BlockSpec((blk, 1), lambda i: (i, 0)),
      ],
      out_specs=[
          pl.BlockSpec((blk, HH), lambda i: (i, 0)),
          pl.BlockSpec((blk, HH), lambda i: (i, 0)),
      ],
      out_shape=[
          jax.ShapeDtypeStruct((NP_, HH), jnp.float32),
          jax.ShapeDtypeStruct((NP_, HH), jnp.float32),
      ],
  )(xpad, wpad, deg.reshape(NP_, 1))


def _edge_body(row_hbm, col_hbm, ew_hbm, xws0_hbm, xws1_hbm, out_hbm,
               rowv, colv, ewv, b0, b1, b2, b3, acc_sh,
               gs0, gs1, gs2, gs3, ss0, ss1, ss2, ss3):
  c = lax.axis_index("c")
  s = lax.axis_index("s")
  bufs = (b0, b1, b2, b3)
  gsems = (gs0, gs1, gs2, gs3)
  ssems = (ss0, ss1, ss2, ss3)

  def _z(e):
    for j in range(HH // 16):
      b0[e, pl.ds(j * 16, 16)] = jnp.zeros((16,), jnp.float32)
  pl.loop(0, ECHUNK)(_z)

  def _zc(i):
    pltpu.sync_copy(
        b0, acc_sh.at[pl.ds(s * NODES_PER_TILE + i * ECHUNK, ECHUNK)])
  pl.loop(0, NODES_PER_TILE // ECHUNK)(_zc)

  plsc.subcore_barrier()

  def _chunks(src_hbm):
    pltpu.sync_copy(row_hbm.at[s, pl.ds(0, SLAB)], rowv.at[0])
    pltpu.async_copy(src_hbm.at[rowv.at[0, 0]], b0, gs0)
    pltpu.async_copy(src_hbm.at[rowv.at[0, 1]], b1, gs1)

    def _body(g):
      for k in range(NBUF):
        c = g + k
        t = c // SLAB
        k8 = c - t * SLAB
        tp = lax.rem(t, 2)

        @pl.when(k8 == 0)
        def _stage():
          pltpu.sync_copy(col_hbm.at[s, pl.ds(t * SLAB, SLAB)], colv.at[tp])
          pltpu.sync_copy(ew_hbm.at[s, pl.ds(t * SLAB, SLAB)], ewv)

          @pl.when(t + 1 < NSLAB)
          def _():
            pltpu.sync_copy(row_hbm.at[s, pl.ds((t + 1) * SLAB, SLAB)],
                            rowv.at[lax.rem(t + 1, 2)])

        pltpu.make_async_copy(src_hbm.at[rowv.at[0, 0]], bufs[k],
                              gsems[k]).wait()

        def _eb(eb):
          wv = ewv[k8, pl.ds(eb * 16, 16)]
          for kk in range(16):
            e = eb * 16 + kk
            w = wv[kk]
            for j in range(HH // 16):
              bufs[k][e, pl.ds(j * 16, 16)] = (
                  bufs[k][e, pl.ds(j * 16, 16)] * w)
        pl.loop(0, ECHUNK // 16)(_eb)

        nb = (k + 2) % NBUF

        @pl.when(c + 2 < NCHUNK)
        def _():
          @pl.when(c >= 2)
          def _():
            pltpu.make_async_copy(bufs[nb], acc_sh.at[colv.at[0, 0]],
                                  ssems[nb]).wait()
          c2 = c + 2
          t2 = c2 // SLAB
          pltpu.async_copy(
              src_hbm.at[rowv.at[lax.rem(t2, 2), c2 - t2 * SLAB]],
              bufs[nb], gsems[nb])

        pltpu.async_copy(bufs[k], acc_sh.at[colv.at[tp, k8]], ssems[k],
                         add=True)

    pl.loop(0, NCHUNK, step=NBUF)(_body)

    for k in range(NBUF):
      pltpu.make_async_copy(bufs[k], acc_sh.at[colv.at[0, 0]],
                            ssems[k]).wait()

  @pl.when(c == 0)
  def _():
    _chunks(xws0_hbm)

  @pl.when(c == 1)
  def _():
    _chunks(xws1_hbm)

  plsc.subcore_barrier()

  nbase = s * NODES_PER_TILE

  @pl.when(c == 0)
  def _():
    pltpu.sync_copy(acc_sh.at[pl.ds(nbase, NODES_PER_TILE)],
                    out_hbm.at[0, pl.ds(nbase, NODES_PER_TILE)])

  @pl.when(c == 1)
  def _():
    pltpu.sync_copy(acc_sh.at[pl.ds(nbase, NODES_PER_TILE)],
                    out_hbm.at[1, pl.ds(nbase, NODES_PER_TILE)])


def _phase_c(row2, col2, ew2, xws0, xws1):
  return pl.kernel(
      _edge_body,
      out_type=jax.ShapeDtypeStruct((2, NP_, HH), jnp.float32),
      mesh=_SC_MESH,
      scratch_types=[
          pltpu.VMEM((2, SLAB, ECHUNK), jnp.int32),
          pltpu.VMEM((2, SLAB, ECHUNK), jnp.int32),
          pltpu.VMEM((SLAB, ECHUNK), jnp.float32),
          pltpu.VMEM((ECHUNK, HH), jnp.float32),
          pltpu.VMEM((ECHUNK, HH), jnp.float32),
          pltpu.VMEM((ECHUNK, HH), jnp.float32),
          pltpu.VMEM((ECHUNK, HH), jnp.float32),
          pltpu.VMEM_SHARED((NP_, HH), jnp.float32),
          pltpu.SemaphoreType.DMA,
          pltpu.SemaphoreType.DMA,
          pltpu.SemaphoreType.DMA,
          pltpu.SemaphoreType.DMA,
          pltpu.SemaphoreType.DMA,
          pltpu.SemaphoreType.DMA,
          pltpu.SemaphoreType.DMA,
          pltpu.SemaphoreType.DMA,
      ],
  )(row2, col2, ew2, xws0, xws1)


def _fc_body(a0_ref, a1_ref, deg_ref, w0_ref, w1_ref, b_ref, o_ref):
  deg = deg_ref[...]
  d = jnp.where(deg > 0.0, lax.rsqrt(jnp.maximum(deg, 1e-12)), 0.0)
  h = jnp.dot(a0_ref[...] * d, w0_ref[...], preferred_element_type=jnp.float32)
  h = h + jnp.dot(a1_ref[...] * d, w1_ref[...],
                  preferred_element_type=jnp.float32)
  o_ref[...] = jnp.tanh(h + b_ref[...])


def _phase_d(acc0, acc1, deg, wf0, wf1, beff):
  blk = 256
  return pl.pallas_call(
      _fc_body,
      grid=(NP_ // blk,),
      in_specs=[
          pl.BlockSpec((blk, HH), lambda i: (i, 0)),
          pl.BlockSpec((blk, HH), lambda i: (i, 0)),
          pl.BlockSpec((blk, 1), lambda i: (i, 0)),
          pl.BlockSpec((HH, O), lambda i: (0, 0)),
          pl.BlockSpec((HH, O), lambda i: (0, 0)),
          pl.BlockSpec((1, O), lambda i: (0, 0)),
      ],
      out_specs=pl.BlockSpec((blk, O), lambda i: (i, 0)),
      out_shape=jax.ShapeDtypeStruct((NP_, O), jnp.float32),
  )(acc0, acc1, deg.reshape(NP_, 1), wf0, wf1, beff.reshape(1, O))


@jax.jit
def kernel(x_features, x_edge_idx, x_edge_attr, W_gcn, b_gcn, W_fc, b_fc):
  row = x_edge_idx[0].astype(jnp.int32)
  col = x_edge_idx[1].astype(jnp.int32)
  loop = jnp.arange(N, dtype=jnp.int32)
  pad = E2P - E2
  row2 = jnp.concatenate([row, loop, jnp.zeros((pad,), jnp.int32)])
  col2 = jnp.concatenate([col, loop, jnp.zeros((pad,), jnp.int32)])
  ew2 = jnp.concatenate([x_edge_attr.astype(jnp.float32),
                         jnp.ones((N,), jnp.float32),
                         jnp.zeros((pad,), jnp.float32)])
  row2 = row2.reshape(NTILES, NCHUNK, ECHUNK)
  col2 = col2.reshape(NTILES, NCHUNK, ECHUNK)
  ew2 = ew2.reshape(NTILES, NCHUNK, ECHUNK)

  deg = _phase_a(col2, ew2)

  xpad = jnp.zeros((NP_, D), jnp.float32).at[:N].set(x_features)
  wpad = jnp.zeros((D, HP), jnp.float32).at[:, :H].set(W_gcn)
  xws0, xws1 = _phase_b(xpad, wpad, deg)

  acc = _phase_c(row2, col2, ew2, xws0, xws1)

  wfpad = jnp.zeros((HP, O), jnp.float32).at[:H].set(W_fc)
  beff = b_gcn @ W_fc + b_fc
  out = _phase_d(acc[0], acc[1], deg, wfpad[:HH], wfpad[HH:], beff)
  return out[:N]


# speedup vs baseline: 9.8516x; 1.0250x over previous
import jax
import jax.numpy as jnp
from jax import lax
from jax.experimental import pallas as pl
from jax.experimental.pallas import tpu as pltpu
from jax.experimental.pallas import tpu_sc as plsc

N = 10000
E = 160000
D = 256
H = 210
O = 128

NP_ = 10240
HP = 256
HH = 128
NTILES = 16
NODES_PER_TILE = NP_ // NTILES
ECHUNK = 80
NCHUNK = 128
NBUF = 4
SLAB = 8
NSLAB = NCHUNK // SLAB
EPT = NCHUNK * ECHUNK
E2P = EPT * NTILES

_SC_MESH = plsc.VectorSubcoreMesh(core_axis_name="c", subcore_axis_name="s")


def _deg_body(col_hbm, ew_hbm, deg_hbm, colv, ewv, degv, deg_sh, sem):
  c = lax.axis_index("c")
  s = lax.axis_index("s")

  @pl.when(c == 0)
  def _():
    def _z(i):
      degv[pl.ds(i * 16, 16)] = jnp.zeros((16,), jnp.float32)
    pl.loop(0, NODES_PER_TILE // 16)(_z)
    pltpu.sync_copy(degv, deg_sh.at[pl.ds(s * NODES_PER_TILE, NODES_PER_TILE)])
    plsc.subcore_barrier()

    pltpu.sync_copy(col_hbm.at[s], colv)
    pltpu.sync_copy(ew_hbm.at[s], ewv)

    def _sc(g):
      pltpu.async_copy(ewv.at[g], deg_sh.at[colv.at[g]], sem, add=True).wait()
    pl.loop(0, NCHUNK)(_sc)
    plsc.subcore_barrier()

    pltpu.sync_copy(deg_sh.at[pl.ds(s * NODES_PER_TILE, NODES_PER_TILE)],
                    deg_hbm.at[pl.ds(s * NODES_PER_TILE, NODES_PER_TILE)])


def _phase_a(col2, ew2):
  return pl.kernel(
      _deg_body,
      out_type=jax.ShapeDtypeStruct((NP_,), jnp.float32),
      mesh=_SC_MESH,
      scratch_types=[
          pltpu.VMEM((NCHUNK, ECHUNK), jnp.int32),
          pltpu.VMEM((NCHUNK, ECHUNK), jnp.float32),
          pltpu.VMEM((NODES_PER_TILE,), jnp.float32),
          pltpu.VMEM_SHARED((NP_,), jnp.float32),
          pltpu.SemaphoreType.DMA,
      ],
  )(col2, ew2)


def _xws_body(x_ref, w_ref, deg_ref, o0_ref, o1_ref):
  xw = jnp.dot(x_ref[...], w_ref[...], preferred_element_type=jnp.float32)
  dis = lax.rsqrt(deg_ref[...] + 1.0)
  xws = xw * dis
  o0_ref[...] = xws[:, :HH]
  o1_ref[...] = xws[:, HH:]


def _phase_b(x, wpad, deg):
  blk = 400
  return pl.pallas_call(
      _xws_body,
      grid=(N // blk,),
      in_specs=[
          pl.BlockSpec((blk, D), lambda i: (i, 0)),
          pl.BlockSpec((D, HP), lambda i: (0, 0)),
          pl.BlockSpec((blk, 1), lambda i: (i, 0)),
      ],
      out_specs=[
          pl.BlockSpec((blk, HH), lambda i: (i, 0)),
          pl.BlockSpec((blk, HH), lambda i: (i, 0)),
      ],
      out_shape=[
          jax.ShapeDtypeStruct((N, HH), jnp.float32),
          jax.ShapeDtypeStruct((N, HH), jnp.float32),
      ],
  )(x, wpad, deg.reshape(N, 1))


def _edge_body(row_hbm, col_hbm, ew_hbm, xws0_hbm, xws1_hbm, out_hbm,
               rowv, colv, ewv, b0, b1, b2, b3, acc_sh,
               gs0, gs1, gs2, gs3, ss0, ss1, ss2, ss3):
  c = lax.axis_index("c")
  s = lax.axis_index("s")
  bufs = (b0, b1, b2, b3)
  gsems = (gs0, gs1, gs2, gs3)
  ssems = (ss0, ss1, ss2, ss3)

  def _z(e):
    for j in range(HH // 16):
      b0[e, pl.ds(j * 16, 16)] = jnp.zeros((16,), jnp.float32)
  pl.loop(0, ECHUNK)(_z)

  def _zc(i):
    pltpu.sync_copy(
        b0, acc_sh.at[pl.ds(s * NODES_PER_TILE + i * ECHUNK, ECHUNK)])
  pl.loop(0, NODES_PER_TILE // ECHUNK)(_zc)

  plsc.subcore_barrier()

  def _chunks(src_hbm):
    pltpu.sync_copy(row_hbm.at[s, pl.ds(0, SLAB)], rowv.at[0])
    pltpu.async_copy(src_hbm.at[rowv.at[0, 0]], b0, gs0)
    pltpu.async_copy(src_hbm.at[rowv.at[0, 1]], b1, gs1)

    def _body(g):
      for k in range(NBUF):
        c = g + k
        t = c // SLAB
        k8 = c - t * SLAB
        tp = lax.rem(t, 2)

        @pl.when(k8 == 0)
        def _stage():
          pltpu.sync_copy(col_hbm.at[s, pl.ds(t * SLAB, SLAB)], colv.at[tp])
          pltpu.sync_copy(ew_hbm.at[s, pl.ds(t * SLAB, SLAB)], ewv)

          @pl.when(t + 1 < NSLAB)
          def _():
            pltpu.sync_copy(row_hbm.at[s, pl.ds((t + 1) * SLAB, SLAB)],
                            rowv.at[lax.rem(t + 1, 2)])

        pltpu.make_async_copy(src_hbm.at[rowv.at[0, 0]], bufs[k],
                              gsems[k]).wait()

        def _eb(eb):
          wv = ewv[k8, pl.ds(eb * 16, 16)]
          for kk in range(16):
            e = eb * 16 + kk
            w = wv[kk]
            for j in range(HH // 16):
              bufs[k][e, pl.ds(j * 16, 16)] = (
                  bufs[k][e, pl.ds(j * 16, 16)] * w)
        pl.loop(0, ECHUNK // 16)(_eb)

        nb = (k + 2) % NBUF

        @pl.when(c + 2 < NCHUNK)
        def _():
          @pl.when(c >= 2)
          def _():
            pltpu.make_async_copy(bufs[nb], acc_sh.at[colv.at[0, 0]],
                                  ssems[nb]).wait()
          c2 = c + 2
          t2 = c2 // SLAB
          pltpu.async_copy(
              src_hbm.at[rowv.at[lax.rem(t2, 2), c2 - t2 * SLAB]],
              bufs[nb], gsems[nb])

        pltpu.async_copy(bufs[k], acc_sh.at[colv.at[tp, k8]], ssems[k],
                         add=True)

    pl.loop(0, NCHUNK, step=NBUF)(_body)

    for k in range(NBUF):
      pltpu.make_async_copy(bufs[k], acc_sh.at[colv.at[0, 0]],
                            ssems[k]).wait()

  @pl.when(c == 0)
  def _():
    _chunks(xws0_hbm)

  @pl.when(c == 1)
  def _():
    _chunks(xws1_hbm)

  plsc.subcore_barrier()

  nbase = s * NODES_PER_TILE

  @pl.when(c == 0)
  def _():
    pltpu.sync_copy(acc_sh.at[pl.ds(nbase, NODES_PER_TILE)],
                    out_hbm.at[0, pl.ds(nbase, NODES_PER_TILE)])

  @pl.when(c == 1)
  def _():
    pltpu.sync_copy(acc_sh.at[pl.ds(nbase, NODES_PER_TILE)],
                    out_hbm.at[1, pl.ds(nbase, NODES_PER_TILE)])


def _phase_c(row2, col2, ew2, xws0, xws1):
  return pl.kernel(
      _edge_body,
      out_type=jax.ShapeDtypeStruct((2, NP_, HH), jnp.float32),
      mesh=_SC_MESH,
      scratch_types=[
          pltpu.VMEM((2, SLAB, ECHUNK), jnp.int32),
          pltpu.VMEM((2, SLAB, ECHUNK), jnp.int32),
          pltpu.VMEM((SLAB, ECHUNK), jnp.float32),
          pltpu.VMEM((ECHUNK, HH), jnp.float32),
          pltpu.VMEM((ECHUNK, HH), jnp.float32),
          pltpu.VMEM((ECHUNK, HH), jnp.float32),
          pltpu.VMEM((ECHUNK, HH), jnp.float32),
          pltpu.VMEM_SHARED((NP_, HH), jnp.float32),
          pltpu.SemaphoreType.DMA,
          pltpu.SemaphoreType.DMA,
          pltpu.SemaphoreType.DMA,
          pltpu.SemaphoreType.DMA,
          pltpu.SemaphoreType.DMA,
          pltpu.SemaphoreType.DMA,
          pltpu.SemaphoreType.DMA,
          pltpu.SemaphoreType.DMA,
      ],
  )(row2, col2, ew2, xws0, xws1)


def _fc_body(a0_ref, a1_ref, x0_ref, x1_ref, deg_ref, w0_ref, w1_ref, b_ref,
             o_ref):
  d = lax.rsqrt(deg_ref[...] + 1.0)
  h = jnp.dot((a0_ref[...] + x0_ref[...]) * d, w0_ref[...],
              preferred_element_type=jnp.float32)
  h = h + jnp.dot((a1_ref[...] + x1_ref[...]) * d, w1_ref[...],
                  preferred_element_type=jnp.float32)
  o_ref[...] = jnp.tanh(h + b_ref[...])


def _phase_d(acc0, acc1, xws0, xws1, deg, wf0, wf1, beff):
  blk = 400
  return pl.pallas_call(
      _fc_body,
      grid=(N // blk,),
      in_specs=[
          pl.BlockSpec((blk, HH), lambda i: (i, 0)),
          pl.BlockSpec((blk, HH), lambda i: (i, 0)),
          pl.BlockSpec((blk, HH), lambda i: (i, 0)),
          pl.BlockSpec((blk, HH), lambda i: (i, 0)),
          pl.BlockSpec((blk, 1), lambda i: (i, 0)),
          pl.BlockSpec((HH, O), lambda i: (0, 0)),
          pl.BlockSpec((HH, O), lambda i: (0, 0)),
          pl.BlockSpec((1, O), lambda i: (0, 0)),
      ],
      out_specs=pl.BlockSpec((blk, O), lambda i: (i, 0)),
      out_shape=jax.ShapeDtypeStruct((N, O), jnp.float32),
  )(acc0, acc1, xws0, xws1, deg.reshape(N, 1), wf0, wf1, beff.reshape(1, O))


@jax.jit
def kernel(x_features, x_edge_idx, x_edge_attr, W_gcn, b_gcn, W_fc, b_fc):
  pad = E2P - E
  row2 = jnp.pad(x_edge_idx[0].astype(jnp.int32),
                 (0, pad)).reshape(NTILES, NCHUNK, ECHUNK)
  col2 = jnp.pad(x_edge_idx[1].astype(jnp.int32),
                 (0, pad)).reshape(NTILES, NCHUNK, ECHUNK)
  ew2 = jnp.pad(x_edge_attr.astype(jnp.float32),
                (0, pad)).reshape(NTILES, NCHUNK, ECHUNK)

  deg = _phase_a(col2, ew2)[:N]

  wpad = jnp.zeros((D, HP), jnp.float32).at[:, :H].set(W_gcn)
  xws0, xws1 = _phase_b(x_features, wpad, deg)

  acc = _phase_c(row2, col2, ew2, xws0, xws1)

  wfpad = jnp.zeros((HP, O), jnp.float32).at[:H].set(W_fc)
  beff = b_gcn @ W_fc + b_fc
  return _phase_d(acc[0], acc[1], xws0, xws1, deg, wfpad[:HH], wfpad[HH:],
                  beff)


# speedup vs baseline: 10.0156x; 1.0167x over previous
import jax
import jax.numpy as jnp
from jax import lax
from jax.experimental import pallas as pl
from jax.experimental.pallas import tpu as pltpu
from jax.experimental.pallas import tpu_sc as plsc

N = 10000
E = 160000
D = 256
H = 210
O = 128

NP_ = 10240
HP = 256
HH = 128
NTILES = 16
NODES_PER_TILE = NP_ // NTILES
ECHUNK = 80
NCHUNK = 128
NBUF = 4
SLAB = 8
NSLAB = NCHUNK // SLAB
EPT = NCHUNK * ECHUNK
E2P = EPT * NTILES

_SC_MESH = plsc.VectorSubcoreMesh(core_axis_name="c", subcore_axis_name="s")


def _deg_body(col_hbm, ew_hbm, deg_hbm, colv, ewv, degv, deg_sh, sem):
  c = lax.axis_index("c")
  s = lax.axis_index("s")

  @pl.when(c == 0)
  def _():
    def _z(i):
      degv[pl.ds(i * 16, 16)] = jnp.zeros((16,), jnp.float32)
    pl.loop(0, NODES_PER_TILE // 16)(_z)
    pltpu.sync_copy(degv, deg_sh.at[pl.ds(s * NODES_PER_TILE, NODES_PER_TILE)])
    plsc.subcore_barrier()

    pltpu.sync_copy(col_hbm.at[s], colv)
    pltpu.sync_copy(ew_hbm.at[s], ewv)

    def _sc(g):
      pltpu.async_copy(ewv.at[g], deg_sh.at[colv.at[g]], sem, add=True).wait()
    pl.loop(0, NCHUNK)(_sc)
    plsc.subcore_barrier()

    pltpu.sync_copy(deg_sh.at[pl.ds(s * NODES_PER_TILE, NODES_PER_TILE)],
                    deg_hbm.at[pl.ds(s * NODES_PER_TILE, NODES_PER_TILE)])


def _phase_a(col2, ew2):
  return pl.kernel(
      _deg_body,
      out_type=jax.ShapeDtypeStruct((NP_,), jnp.float32),
      mesh=_SC_MESH,
      scratch_types=[
          pltpu.VMEM((NCHUNK, ECHUNK), jnp.int32),
          pltpu.VMEM((NCHUNK, ECHUNK), jnp.float32),
          pltpu.VMEM((NODES_PER_TILE,), jnp.float32),
          pltpu.VMEM_SHARED((NP_,), jnp.float32),
          pltpu.SemaphoreType.DMA,
      ],
  )(col2, ew2)


def _xws_body(x_ref, w_ref, deg_ref, o0_ref, o1_ref):
  xw = jnp.dot(x_ref[...], w_ref[...], preferred_element_type=jnp.float32)
  dis = lax.rsqrt(deg_ref[...] + 1.0)
  xws = xw * dis
  o0_ref[...] = xws[:, :HH]
  o1_ref[...] = xws[:, HH:]


def _phase_b(x, wpad, deg):
  blk = 400
  return pl.pallas_call(
      _xws_body,
      grid=(N // blk,),
      in_specs=[
          pl.BlockSpec((blk, D), lambda i: (i, 0)),
          pl.BlockSpec((D, HP), lambda i: (0, 0)),
          pl.BlockSpec((blk, 1), lambda i: (i, 0)),
      ],
      out_specs=[
          pl.BlockSpec((blk, HH), lambda i: (i, 0)),
          pl.BlockSpec((blk, HH), lambda i: (i, 0)),
      ],
      out_shape=[
          jax.ShapeDtypeStruct((N, HH), jnp.float32),
          jax.ShapeDtypeStruct((N, HH), jnp.float32),
      ],
  )(x, wpad, deg.reshape(N, 1))


def _edge_body(row_hbm, col_hbm, ew_hbm, xws0_hbm, xws1_hbm, out_hbm,
               rowv, colv, ewv, b0, b1, b2, b3, acc_sh,
               gs0, gs1, gs2, gs3, ss0, ss1, ss2, ss3, stsem):
  c = lax.axis_index("c")
  s = lax.axis_index("s")
  bufs = (b0, b1, b2, b3)
  gsems = (gs0, gs1, gs2, gs3)
  ssems = (ss0, ss1, ss2, ss3)

  def _z(e):
    for j in range(HH // 16):
      b0[e, pl.ds(j * 16, 16)] = jnp.zeros((16,), jnp.float32)
  pl.loop(0, ECHUNK)(_z)

  def _zc(i):
    pltpu.sync_copy(
        b0, acc_sh.at[pl.ds(s * NODES_PER_TILE + i * ECHUNK, ECHUNK)])
  pl.loop(0, NODES_PER_TILE // ECHUNK)(_zc)

  plsc.subcore_barrier()

  def _chunks(src_hbm):
    pltpu.sync_copy(row_hbm.at[s, pl.ds(0, SLAB)], rowv.at[0])
    pltpu.sync_copy(col_hbm.at[s, pl.ds(0, SLAB)], colv.at[0])
    pltpu.sync_copy(ew_hbm.at[s, pl.ds(0, SLAB)], ewv.at[0])
    pltpu.async_copy(src_hbm.at[rowv.at[0, 0]], b0, gs0)
    pltpu.async_copy(src_hbm.at[rowv.at[0, 1]], b1, gs1)

    def _body(g):
      for k in range(NBUF):
        c = g + k
        t = c // SLAB
        k8 = c - t * SLAB
        tp = lax.rem(t, 2)
        tp1 = lax.rem(t + 1, 2)

        @pl.when((k8 == 2) & (t + 1 < NSLAB))
        def _stage():
          pltpu.async_copy(row_hbm.at[s, pl.ds((t + 1) * SLAB, SLAB)],
                           rowv.at[tp1], stsem)
          pltpu.async_copy(col_hbm.at[s, pl.ds((t + 1) * SLAB, SLAB)],
                           colv.at[tp1], stsem)
          pltpu.async_copy(ew_hbm.at[s, pl.ds((t + 1) * SLAB, SLAB)],
                           ewv.at[tp1], stsem)

        @pl.when((k8 == 6) & (t + 1 < NSLAB))
        def _stage_wait():
          pltpu.make_async_copy(row_hbm.at[s, pl.ds(0, SLAB)],
                                rowv.at[tp1], stsem).wait()
          pltpu.make_async_copy(col_hbm.at[s, pl.ds(0, SLAB)],
                                colv.at[tp1], stsem).wait()
          pltpu.make_async_copy(ew_hbm.at[s, pl.ds(0, SLAB)],
                                ewv.at[tp1], stsem).wait()

        pltpu.make_async_copy(src_hbm.at[rowv.at[0, 0]], bufs[k],
                              gsems[k]).wait()

        def _eb(eb):
          wv = ewv[tp, k8, pl.ds(eb * 16, 16)]
          for kk in range(16):
            e = eb * 16 + kk
            w = wv[kk]
            for j in range(HH // 16):
              bufs[k][e, pl.ds(j * 16, 16)] = (
                  bufs[k][e, pl.ds(j * 16, 16)] * w)
        pl.loop(0, ECHUNK // 16)(_eb)

        nb = (k + 2) % NBUF

        @pl.when(c + 2 < NCHUNK)
        def _():
          @pl.when(c >= 2)
          def _():
            pltpu.make_async_copy(bufs[nb], acc_sh.at[colv.at[0, 0]],
                                  ssems[nb]).wait()
          c2 = c + 2
          t2 = c2 // SLAB
          pltpu.async_copy(
              src_hbm.at[rowv.at[lax.rem(t2, 2), c2 - t2 * SLAB]],
              bufs[nb], gsems[nb])

        pltpu.async_copy(bufs[k], acc_sh.at[colv.at[tp, k8]], ssems[k],
                         add=True)

    pl.loop(0, NCHUNK, step=NBUF)(_body)

    for k in range(NBUF):
      pltpu.make_async_copy(bufs[k], acc_sh.at[colv.at[0, 0]],
                            ssems[k]).wait()

  @pl.when(c == 0)
  def _():
    _chunks(xws0_hbm)

  @pl.when(c == 1)
  def _():
    _chunks(xws1_hbm)

  plsc.subcore_barrier()

  nbase = s * NODES_PER_TILE

  @pl.when(c == 0)
  def _():
    pltpu.sync_copy(acc_sh.at[pl.ds(nbase, NODES_PER_TILE)],
                    out_hbm.at[0, pl.ds(nbase, NODES_PER_TILE)])

  @pl.when(c == 1)
  def _():
    pltpu.sync_copy(acc_sh.at[pl.ds(nbase, NODES_PER_TILE)],
                    out_hbm.at[1, pl.ds(nbase, NODES_PER_TILE)])


def _phase_c(row2, col2, ew2, xws0, xws1):
  return pl.kernel(
      _edge_body,
      out_type=jax.ShapeDtypeStruct((2, NP_, HH), jnp.float32),
      mesh=_SC_MESH,
      scratch_types=[
          pltpu.VMEM((2, SLAB, ECHUNK), jnp.int32),
          pltpu.VMEM((2, SLAB, ECHUNK), jnp.int32),
          pltpu.VMEM((2, SLAB, ECHUNK), jnp.float32),
          pltpu.VMEM((ECHUNK, HH), jnp.float32),
          pltpu.VMEM((ECHUNK, HH), jnp.float32),
          pltpu.VMEM((ECHUNK, HH), jnp.float32),
          pltpu.VMEM((ECHUNK, HH), jnp.float32),
          pltpu.VMEM_SHARED((NP_, HH), jnp.float32),
          pltpu.SemaphoreType.DMA,
          pltpu.SemaphoreType.DMA,
          pltpu.SemaphoreType.DMA,
          pltpu.SemaphoreType.DMA,
          pltpu.SemaphoreType.DMA,
          pltpu.SemaphoreType.DMA,
          pltpu.SemaphoreType.DMA,
          pltpu.SemaphoreType.DMA,
          pltpu.SemaphoreType.DMA,
      ],
  )(row2, col2, ew2, xws0, xws1)


def _fc_body(a0_ref, a1_ref, x0_ref, x1_ref, deg_ref, w0_ref, w1_ref, b_ref,
             o_ref):
  d = lax.rsqrt(deg_ref[...] + 1.0)
  h = jnp.dot((a0_ref[...] + x0_ref[...]) * d, w0_ref[...],
              preferred_element_type=jnp.float32)
  h = h + jnp.dot((a1_ref[...] + x1_ref[...]) * d, w1_ref[...],
                  preferred_element_type=jnp.float32)
  o_ref[...] = jnp.tanh(h + b_ref[...])


def _phase_d(acc0, acc1, xws0, xws1, deg, wf0, wf1, beff):
  blk = 400
  return pl.pallas_call(
      _fc_body,
      grid=(N // blk,),
      in_specs=[
          pl.BlockSpec((blk, HH), lambda i: (i, 0)),
          pl.BlockSpec((blk, HH), lambda i: (i, 0)),
          pl.BlockSpec((blk, HH), lambda i: (i, 0)),
          pl.BlockSpec((blk, HH), lambda i: (i, 0)),
          pl.BlockSpec((blk, 1), lambda i: (i, 0)),
          pl.BlockSpec((HH, O), lambda i: (0, 0)),
          pl.BlockSpec((HH, O), lambda i: (0, 0)),
          pl.BlockSpec((1, O), lambda i: (0, 0)),
      ],
      out_specs=pl.BlockSpec((blk, O), lambda i: (i, 0)),
      out_shape=jax.ShapeDtypeStruct((N, O), jnp.float32),
  )(acc0, acc1, xws0, xws1, deg.reshape(N, 1), wf0, wf1, beff.reshape(1, O))


@jax.jit
def kernel(x_features, x_edge_idx, x_edge_attr, W_gcn, b_gcn, W_fc, b_fc):
  pad = E2P - E
  row2 = jnp.pad(x_edge_idx[0].astype(jnp.int32),
                 (0, pad)).reshape(NTILES, NCHUNK, ECHUNK)
  col2 = jnp.pad(x_edge_idx[1].astype(jnp.int32),
                 (0, pad)).reshape(NTILES, NCHUNK, ECHUNK)
  ew2 = jnp.pad(x_edge_attr.astype(jnp.float32),
                (0, pad)).reshape(NTILES, NCHUNK, ECHUNK)

  deg = _phase_a(col2, ew2)[:N]

  wpad = jnp.zeros((D, HP), jnp.float32).at[:, :H].set(W_gcn)
  xws0, xws1 = _phase_b(x_features, wpad, deg)

  acc = _phase_c(row2, col2, ew2, xws0, xws1)

  wfpad = jnp.zeros((HP, O), jnp.float32).at[:H].set(W_fc)
  beff = b_gcn @ W_fc + b_fc
  return _phase_d(acc[0], acc[1], xws0, xws1, deg, wfpad[:HH], wfpad[HH:],
                  beff)
